# Initial kernel scaffold; baseline (speedup 1.0000x reference)
import jax
import jax.numpy as jnp
from jax.experimental import pallas as pl


def kernel(x, edge_attr, edge_index, hproj_w, hproj_b, eproj_w, eproj_b, U_w, U_b, V_w, V_b, A_w, A_b, B_w, B_b, C_w, C_b, hbn_g, hbn_b, ebn_g, ebn_b, fe_w1, fe_b1, fe_w2, fe_b2, fe_w3, fe_b3):
    raise NotImplementedError("write your pallas kernel here")


# speedup vs baseline: 2.2203x; 2.2203x over previous
import functools

import jax
import jax.numpy as jnp
from jax import lax
from jax.experimental import pallas as pl
from jax.experimental.pallas import tpu as pltpu
from jax.experimental.pallas import tpu_sc as plsc

_N = 10000
_E = 320000
_D = 128
_EPS = 1e-5

_BE = 4000
_GE = _E // _BE

_NC = 2
_NS = 16
_NW = _NC * _NS
_EPW = _E // _NW
_CB = 80
_NCH = _EPW // _CB
_RPT = 640
_RCH = 16
_CW = 128


def _silu(v):
    return v * jax.nn.sigmoid(v)


def _matT(a, w):
    return lax.dot_general(a, w, (((1,), (1,)), ((), ())),
                           preferred_element_type=jnp.float32)


def _hproj_body(x_ref, w_ref, b_ref, o_ref):
    o_ref[...] = _silu(_matT(x_ref[...], w_ref[...]) + b_ref[...])


def _hproj(x, w, b):
    return pl.pallas_call(
        _hproj_body,
        out_shape=jax.ShapeDtypeStruct((_N, _D), jnp.float32),
    )(x, w, b)


def _node_pre_body(h_ref, uw, ub, vw, vb, bw, bb, cw, cb, uo, vo, bo, co):
    h = h_ref[...]
    uo[...] = _matT(h, uw[...]) + ub[...]
    vo[...] = _matT(h, vw[...]) + vb[...]
    bo[...] = _matT(h, bw[...]) + bb[...]
    co[...] = _matT(h, cw[...]) + cb[...]


def _node_pre(h, uw, ub, vw, vb, bw, bb, cw, cb):
    return pl.pallas_call(
        _node_pre_body,
        out_shape=[jax.ShapeDtypeStruct((_N, _D), jnp.float32)] * 4,
    )(h, uw, ub, vw, vb, bw, bb, cw, cb)


def _f0_body(ea_ref, ew_ref, eb_ref, e_ref, sg_ref):
    e = _silu(ea_ref[...] * ew_ref[...] + eb_ref[...])
    e_ref[...] = e
    sg_ref[...] = jax.nn.sigmoid(e)


def _f0(ea, ew, eb):
    return pl.pallas_call(
        _f0_body,
        grid=(_GE,),
        in_specs=[
            pl.BlockSpec((_BE, 1), lambda i: (i, 0)),
            pl.BlockSpec((1, _D), lambda i: (0, 0)),
            pl.BlockSpec((1, _D), lambda i: (0, 0)),
        ],
        out_specs=[pl.BlockSpec((_BE, _D), lambda i: (i, 0))] * 2,
        out_shape=[jax.ShapeDtypeStruct((_E, _D), jnp.float32)] * 2,
    )(ea, ew, eb)


def _bnstats_body(e_ref, g_ref, aw_ref, ab_ref, t_ref, s1_ref, s2_ref):
    t = _matT(e_ref[...], aw_ref[...]) + ab_ref[...] + g_ref[...]
    t_ref[...] = t

    @pl.when(pl.program_id(0) == 0)
    def _():
        s1_ref[...] = jnp.zeros_like(s1_ref)
        s2_ref[...] = jnp.zeros_like(s2_ref)

    s1_ref[...] += jnp.sum(t, axis=0, keepdims=True)
    s2_ref[...] += jnp.sum(t * t, axis=0, keepdims=True)


def _bnstats(e, g, aw, ab):
    return pl.pallas_call(
        _bnstats_body,
        grid=(_GE,),
        in_specs=[
            pl.BlockSpec((_BE, _D), lambda i: (i, 0)),
            pl.BlockSpec((_BE, _D), lambda i: (i, 0)),
            pl.BlockSpec((_D, _D), lambda i: (0, 0)),
            pl.BlockSpec((1, _D), lambda i: (0, 0)),
        ],
        out_specs=[
            pl.BlockSpec((_BE, _D), lambda i: (i, 0)),
            pl.BlockSpec((1, _D), lambda i: (0, 0)),
            pl.BlockSpec((1, _D), lambda i: (0, 0)),
        ],
        out_shape=[
            jax.ShapeDtypeStruct((_E, _D), jnp.float32),
            jax.ShapeDtypeStruct((1, _D), jnp.float32),
            jax.ShapeDtypeStruct((1, _D), jnp.float32),
        ],
    )(e, g, aw, ab)


def _denom_body(c_ref, o_ref):
    c = c_ref[...]
    s = c[0] + c[1]
    o_ref[...] = jnp.maximum(s[:, 0:1], 1.0)


def _denom(cntp):
    return pl.pallas_call(
        _denom_body,
        out_shape=jax.ShapeDtypeStruct((_N, 1), jnp.float32),
    )(cntp)


def _node_upd_body(h_ref, uh_ref, ap_ref, dn_ref, g_ref, b_ref, o_ref):
    ap = ap_ref[...]
    agg = (ap[0] + ap[1]) / dn_ref[...]
    hu = uh_ref[...] + agg
    mu = jnp.mean(hu, axis=0, keepdims=True)
    var = jnp.mean((hu - mu) * (hu - mu), axis=0, keepdims=True)
    hn = g_ref[...] * (hu - mu) * lax.rsqrt(var + _EPS) + b_ref[...]
    o_ref[...] = h_ref[...] + _silu(hn)


def _node_upd(h, uh, aggp, denom, g, b):
    return pl.pallas_call(
        _node_upd_body,
        out_shape=jax.ShapeDtypeStruct((_N, _D), jnp.float32),
    )(h, uh, aggp, denom, g, b)


def _ebn_apply(t, s1, s2, g, b):
    mu = s1 * (1.0 / _E)
    var = s2 * (1.0 / _E) - mu * mu
    return g * (t - mu) * lax.rsqrt(var + _EPS) + b


def _fi_body(e_ref, t_ref, s1_ref, s2_ref, g_ref, b_ref, eo_ref, sg_ref):
    tn = _ebn_apply(t_ref[...], s1_ref[...], s2_ref[...], g_ref[...], b_ref[...])
    e = e_ref[...] + _silu(tn)
    eo_ref[...] = e
    sg_ref[...] = jax.nn.sigmoid(e)


def _fi(e, t, s1, s2, g, b):
    return pl.pallas_call(
        _fi_body,
        grid=(_GE,),
        in_specs=[
            pl.BlockSpec((_BE, _D), lambda i: (i, 0)),
            pl.BlockSpec((_BE, _D), lambda i: (i, 0)),
            pl.BlockSpec((1, _D), lambda i: (0, 0)),
            pl.BlockSpec((1, _D), lambda i: (0, 0)),
            pl.BlockSpec((1, _D), lambda i: (0, 0)),
            pl.BlockSpec((1, _D), lambda i: (0, 0)),
        ],
        out_specs=[pl.BlockSpec((_BE, _D), lambda i: (i, 0))] * 2,
        out_shape=[jax.ShapeDtypeStruct((_E, _D), jnp.float32)] * 2,
    )(e, t, s1, s2, g, b)


def _final_body(e_ref, t_ref, s1_ref, s2_ref, g_ref, b_ref,
                w1_ref, b1_ref, w2_ref, b2_ref, w3_ref, b3_ref, z_ref):
    tn = _ebn_apply(t_ref[...], s1_ref[...], s2_ref[...], g_ref[...], b_ref[...])
    e = e_ref[...] + _silu(tn)
    z = _silu(_matT(e, w1_ref[...]) + b1_ref[...])
    z = _silu(_matT(z, w2_ref[...]) + b2_ref[...])
    z3 = jnp.sum(z * w3_ref[...], axis=1, keepdims=True) + b3_ref[0, 0]
    z_ref[...] = jax.nn.sigmoid(z3)


def _final(e, t, s1, s2, g, b, w1, b1, w2, b2, w3, b3):
    return pl.pallas_call(
        _final_body,
        grid=(_GE,),
        in_specs=[
            pl.BlockSpec((_BE, _D), lambda i: (i, 0)),
            pl.BlockSpec((_BE, _D), lambda i: (i, 0)),
            pl.BlockSpec((1, _D), lambda i: (0, 0)),
            pl.BlockSpec((1, _D), lambda i: (0, 0)),
            pl.BlockSpec((1, _D), lambda i: (0, 0)),
            pl.BlockSpec((1, _D), lambda i: (0, 0)),
            pl.BlockSpec((_D, _D), lambda i: (0, 0)),
            pl.BlockSpec((1, _D), lambda i: (0, 0)),
            pl.BlockSpec((_D, _D), lambda i: (0, 0)),
            pl.BlockSpec((1, _D), lambda i: (0, 0)),
            pl.BlockSpec((1, _D), lambda i: (0, 0)),
            pl.BlockSpec((1, 1), lambda i: (0, 0)),
        ],
        out_specs=pl.BlockSpec((_BE, 1), lambda i: (i, 0)),
        out_shape=jax.ShapeDtypeStruct((_E, 1), jnp.float32),
    )(e, t, s1, s2, g, b, w1, b1, w2, b2, w3, b3)


def _sc_pass(sig_e, vh, bh, ch, src, dst):
    mesh = plsc.VectorSubcoreMesh(core_axis_name="c", subcore_axis_name="s",
                                  num_cores=_NC, num_subcores=_NS)

    out_type = [
        jax.ShapeDtypeStruct((_E, _D), jnp.float32),
        jax.ShapeDtypeStruct((_NC * _N, _D), jnp.float32),
    ]
    scratch = [
        pltpu.VMEM((1, _CB), jnp.int32),
        pltpu.VMEM((1, _CB), jnp.int32),
        pltpu.VMEM((_CB, _D), jnp.float32),
        pltpu.VMEM((_CB, _D), jnp.float32),
        pltpu.VMEM((_CB, _D), jnp.float32),
        pltpu.VMEM((_CB, _D), jnp.float32),
        pltpu.VMEM((_RCH, _D), jnp.float32),
        pltpu.VMEM_SHARED((_N, _D), jnp.float32),
        pltpu.SemaphoreType.DMA,
    ]

    def body(sig_h, vh_h, bh_h, ch_h, src_h, dst_h, g_h, agg_h,
             i_s, i_d, a_v, b_v, c_v, d_v, zv, acc, sem):
        cid = lax.axis_index("c")
        sid = lax.axis_index("s")
        wid = cid * _NS + sid

        def zrow(r, _):
            for j in range(_D // 16):
                zv[r, pl.ds(j * 16, 16)] = jnp.zeros((16,), jnp.float32)
            return 0
        lax.fori_loop(0, _RCH, zrow, 0)

        for k in range(_RPT // _RCH):
            off = sid * _RPT + k * _RCH

            @pl.when(off < _N)
            def _():
                pltpu.sync_copy(zv, acc.at[pl.ds(off, _RCH)])

        plsc.subcore_barrier()

        def chunk(k, _):
            row = wid * _NCH + k
            off = row * _CB
            pltpu.sync_copy(src_h.at[pl.ds(row, 1)], i_s)
            pltpu.sync_copy(dst_h.at[pl.ds(row, 1)], i_d)
            cp1 = pltpu.async_copy(sig_h.at[pl.ds(off, _CB)], a_v, sem)
            cp2 = pltpu.async_copy(vh_h.at[i_d.at[0]], b_v, sem)
            cp3 = pltpu.async_copy(bh_h.at[i_s.at[0]], c_v, sem)
            cp4 = pltpu.async_copy(ch_h.at[i_d.at[0]], d_v, sem)
            cp1.wait()
            cp2.wait()
            cp3.wait()
            cp4.wait()

            def comp(r, _):
                for j in range(_D // 16):
                    s = pl.ds(j * 16, 16)
                    b_v[r, s] = a_v[r, s] * b_v[r, s]
                    c_v[r, s] = c_v[r, s] + d_v[r, s]
                return 0
            lax.fori_loop(0, _CB, comp, 0)

            pltpu.sync_copy(c_v, g_h.at[pl.ds(off, _CB)])
            pltpu.sync_copy(b_v, acc.at[i_s.at[0]], add=True)
            return 0
        lax.fori_loop(0, _NCH, chunk, 0)

        plsc.subcore_barrier()
        for k in range(_RPT // _RCH):
            off = sid * _RPT + k * _RCH

            @pl.when(off < _N)
            def _():
                pltpu.sync_copy(acc.at[pl.ds(off, _RCH)], zv)
                pltpu.sync_copy(zv, agg_h.at[pl.ds(cid * _N + off, _RCH)])

    f = pl.kernel(body, out_type=out_type, mesh=mesh, scratch_types=scratch)
    return f(sig_e, vh, bh, ch,
             src.reshape(_E // _CB, _CB), dst.reshape(_E // _CB, _CB))


def _sc_counts(src):
    mesh = plsc.VectorSubcoreMesh(core_axis_name="c", subcore_axis_name="s",
                                  num_cores=_NC, num_subcores=_NS)

    out_type = [jax.ShapeDtypeStruct((_NC * _N, _CW), jnp.float32)]
    scratch = [
        pltpu.VMEM((1, _CB), jnp.int32),
        pltpu.VMEM((_CB, _CW), jnp.float32),
        pltpu.VMEM((_RCH, _CW), jnp.float32),
        pltpu.VMEM_SHARED((_N, _CW), jnp.float32),
    ]

    def body(src_h, cnt_h, i_s, ones_v, z2v, cacc):
        cid = lax.axis_index("c")
        sid = lax.axis_index("s")
        wid = cid * _NS + sid

        def zrow2(r, _):
            for j in range(_CW // 16):
                z2v[r, pl.ds(j * 16, 16)] = jnp.zeros((16,), jnp.float32)
            return 0
        lax.fori_loop(0, _RCH, zrow2, 0)

        def orow(r, _):
            for j in range(_CW // 16):
                ones_v[r, pl.ds(j * 16, 16)] = jnp.ones((16,), jnp.float32)
            return 0
        lax.fori_loop(0, _CB, orow, 0)

        for k in range(_RPT // _RCH):
            off = sid * _RPT + k * _RCH

            @pl.when(off < _N)
            def _():
                pltpu.sync_copy(z2v, cacc.at[pl.ds(off, _RCH)])

        plsc.subcore_barrier()

        def chunk(k, _):
            row = wid * _NCH + k
            pltpu.sync_copy(src_h.at[pl.ds(row, 1)], i_s)
            pltpu.sync_copy(ones_v, cacc.at[i_s.at[0]], add=True)
            return 0
        lax.fori_loop(0, _NCH, chunk, 0)

        plsc.subcore_barrier()
        for k in range(_RPT // _RCH):
            off = sid * _RPT + k * _RCH

            @pl.when(off < _N)
            def _():
                pltpu.sync_copy(cacc.at[pl.ds(off, _RCH)], z2v)
                pltpu.sync_copy(z2v, cnt_h.at[pl.ds(cid * _N + off, _RCH)])

    f = pl.kernel(body, out_type=out_type, mesh=mesh, scratch_types=scratch)
    return f(src.reshape(_E // _CB, _CB))[0]


def kernel(x, edge_attr, edge_index, hproj_w, hproj_b, eproj_w, eproj_b,
           U_w, U_b, V_w, V_b, A_w, A_b, B_w, B_b, C_w, C_b,
           hbn_g, hbn_b, ebn_g, ebn_b,
           fe_w1, fe_b1, fe_w2, fe_b2, fe_w3, fe_b3):
    src = edge_index[0]
    dst = edge_index[1]

    r = lambda v: v.reshape(1, _D)
    h = _hproj(x, hproj_w, hproj_b.reshape(1, _D))
    e, sig_e = _f0(edge_attr, eproj_w.reshape(1, _D), r(eproj_b))

    denom = None
    z = None
    for i in range(3):
        Uh, Vh, Bh, Ch = _node_pre(h, U_w[i], r(U_b[i]), V_w[i], r(V_b[i]),
                                   B_w[i], r(B_b[i]), C_w[i], r(C_b[i]))
        if i == 0:
            cntp = _sc_counts(src)
            denom = _denom(cntp.reshape(_NC, _N, _CW))
        g, aggp = _sc_pass(sig_e, Vh, Bh, Ch, src, dst)
        aggp = aggp.reshape(_NC, _N, _D)
        t, s1, s2 = _bnstats(e, g, A_w[i], r(A_b[i]))
        h = _node_upd(h, Uh, aggp, denom, r(hbn_g[i]), r(hbn_b[i]))
        if i < 2:
            e, sig_e = _fi(e, t, s1, s2, r(ebn_g[i]), r(ebn_b[i]))
        else:
            z = _final(e, t, s1, s2, r(ebn_g[i]), r(ebn_b[i]),
                       fe_w1, r(fe_b1), fe_w2, r(fe_b2),
                       fe_w3.reshape(1, _D), fe_b3.reshape(1, 1))
    return z
